# baseline (speedup 1.0000x reference)
import functools

import jax
import jax.numpy as jnp
from jax import lax
from jax.experimental import pallas as pl
from jax.experimental.pallas import tpu as pltpu
from jax.experimental.pallas import tpu_sc as plsc

_NC = 2
_NS = 16
_NW = _NC * _NS

_C = 20000

_BLK = 1280
_CH = 400
_ZB = 128
_LA = 16
_RS = 8
_RB = 32


def _sc_mesh():
    return plsc.VectorSubcoreMesh(
        core_axis_name="c", subcore_axis_name="s",
        num_cores=_NC, num_subcores=_NS)


def _ln(h, g, be):
    mu = jnp.mean(h, axis=-1, keepdims=True)
    var = jnp.mean((h - mu) ** 2, axis=-1, keepdims=True)
    return (h - mu) * lax.rsqrt(var + 1e-5) * g + be


def _mlp_first_body(x_ref, w_ref, b_ref, g_ref, be_ref, o_ref):
    h = jnp.dot(x_ref[...], w_ref[...], preferred_element_type=jnp.float32)
    h = h + b_ref[...]
    o_ref[...] = jnp.maximum(_ln(h, g_ref[...], be_ref[...]), 0.0)


def _mlp_pair_body(h_ref, bc_ref, wt_ref, wb_ref, b_ref, g_ref, be_ref, o_ref,
                   *, ln_relu):
    t = jnp.dot(h_ref[...], wt_ref[...], preferred_element_type=jnp.float32)
    t = t + jnp.dot(bc_ref[...], wb_ref[...], preferred_element_type=jnp.float32)
    t = t + b_ref[...]
    if ln_relu:
        t = jnp.maximum(_ln(t, g_ref[...], be_ref[...]), 0.0)
    o_ref[...] = t


def _mlp_first(x, w, b, g, be):
    n, din = x.shape
    h = w.shape[1]
    return pl.pallas_call(
        _mlp_first_body,
        grid=(n // _BLK,),
        in_specs=[
            pl.BlockSpec((_BLK, din), lambda i: (i, 0)),
            pl.BlockSpec((din, h), lambda i: (0, 0)),
            pl.BlockSpec((1, h), lambda i: (0, 0)),
            pl.BlockSpec((1, h), lambda i: (0, 0)),
            pl.BlockSpec((1, h), lambda i: (0, 0)),
        ],
        out_specs=pl.BlockSpec((_BLK, h), lambda i: (i, 0)),
        out_shape=jax.ShapeDtypeStruct((n, h), jnp.float32),
    )(x, w, b.reshape(1, -1), g.reshape(1, -1), be.reshape(1, -1))


def _mlp_pair(hin, bc, wt, wb, b, g, be, ln_relu):
    n, h = hin.shape
    bw = bc.shape[1]
    return pl.pallas_call(
        functools.partial(_mlp_pair_body, ln_relu=ln_relu),
        grid=(n // _BLK,),
        in_specs=[
            pl.BlockSpec((_BLK, h), lambda i: (i, 0)),
            pl.BlockSpec((_BLK, bw), lambda i: (i, 0)),
            pl.BlockSpec((h, h), lambda i: (0, 0)),
            pl.BlockSpec((bw, h), lambda i: (0, 0)),
            pl.BlockSpec((1, h), lambda i: (0, 0)),
            pl.BlockSpec((1, h), lambda i: (0, 0)),
            pl.BlockSpec((1, h), lambda i: (0, 0)),
        ],
        out_specs=pl.BlockSpec((_BLK, h), lambda i: (i, 0)),
        out_shape=jax.ShapeDtypeStruct((n, h), jnp.float32),
    )(hin, bc, wt, wb, b.reshape(1, -1), g.reshape(1, -1), be.reshape(1, -1))


def _normalize(a):
    c, h = a.shape

    def body(a_ref, o_ref):
        v = a_ref[...]
        nrm = jnp.sqrt(jnp.sum(v * v, axis=-1, keepdims=True))
        o_ref[...] = v / jnp.maximum(nrm, 1e-12)

    return pl.pallas_call(
        body, out_shape=jax.ShapeDtypeStruct((c, h), jnp.float32))(a)


def _prefix_count(eqi):
    pc = jnp.int32(0)
    ok = jnp.int32(1)
    for j in range(16):
        ok = ok * eqi[j]
        pc = pc + ok
    return pc


def _suffix_count(eqi):
    pc = jnp.int32(0)
    ok = jnp.int32(1)
    for j in range(15, -1, -1):
        ok = ok * eqi[j]
        pc = pc + ok
    return pc


def _seg_bcast(hmat, cl):
    n, hd = hmat.shape
    r_per = n // _NW
    nk = hd // 16
    nch = r_per // _CH
    rrow = 16 * hd

    @functools.partial(
        pl.kernel,
        out_type=jax.ShapeDtypeStruct((n * hd,), jnp.float32),
        mesh=_sc_mesh(),
        scratch_types=[
            pltpu.VMEM((2 * _CH,), jnp.int32),
            pltpu.VMEM((2 * _CH * hd,), jnp.float32),
            pltpu.VMEM((hd,), jnp.float32),
            pltpu.VMEM((_RB * 16 * hd,), jnp.float32),
            pltpu.VMEM((16,), jnp.int32),
            pltpu.VMEM((16,), jnp.int32),
            pltpu.VMEM((64,), jnp.int32),
            pltpu.VMEM((_LA * hd,), jnp.float32),
            pltpu.SemaphoreType.DMA,
            pltpu.SemaphoreType.DMA,
        ],
    )
    def k(h_hbm, cl_hbm, bc_hbm, idv, hv, accv, repv, bidv, bidv2, laidv,
          lahv, semc, semb):
        wid = lax.axis_index("s") * _NC + lax.axis_index("c")
        row0 = wid * r_per
        rend = row0 + r_per

        pltpu.sync_copy(
            cl_hbm.at[pl.ds(pl.multiple_of(jnp.maximum(row0 - 16, 0), 16), 16)],
            bidv)
        prev_id = jnp.where(wid > 0, bidv[...][15], -1)
        pltpu.sync_copy(cl_hbm.at[pl.ds(pl.multiple_of(row0, 16), 16)], bidv)
        first_id = bidv[...][0]
        continued = jnp.logical_and(wid > 0, prev_id == first_id)

        pltpu.async_copy(
            cl_hbm.at[pl.ds(pl.multiple_of(row0, 16), _CH)],
            idv.at[0:_CH], semc)
        pltpu.async_copy(
            h_hbm.at[pl.ds(pl.multiple_of(row0 * hd, 8), _CH * hd)],
            hv.at[0:_CH * hd], semc)

        bbase = jnp.maximum(row0 - 64, 0)
        pltpu.sync_copy(
            cl_hbm.at[pl.ds(pl.multiple_of(bbase, 16), 64)], laidv)
        b_cnt = jnp.int32(0)
        still_b = continued
        for kg in (3, 2, 1, 0):
            eqi = jnp.where(laidv[pl.ds(kg * 16, 16)] == prev_id, 1, 0)
            pc = _suffix_count(eqi)
            b_cnt = jnp.where(still_b, b_cnt + pc, b_cnt)
            still_b = jnp.logical_and(
                still_b,
                jnp.logical_and(pc == 16, row0 - (4 - kg) * 16 > 0))

        trip_b = jnp.where(still_b, (row0 - 64) // 16, 0)

        def bext_body(i, st):
            cnt, stl = st
            base = row0 - 64 - (i + 1) * 16
            pltpu.sync_copy(
                cl_hbm.at[pl.ds(pl.multiple_of(base, 16), 16)], bidv2)
            pc = _suffix_count(jnp.where(bidv2[...] == prev_id, 1, 0))
            cnt = jnp.where(stl, cnt + pc, cnt)
            stl = jnp.logical_and(stl, jnp.logical_and(pc == 16, base > 0))
            return (cnt, stl)

        b_cnt, _ = lax.fori_loop(0, trip_b, bext_body, (b_cnt, still_b))

        @pl.when(continued)
        def _():
            for kk in range(nk):
                accv[pl.ds(kk * 16, 16)] = jnp.full(
                    (16,), -jnp.inf, jnp.float32)

        def bfold_body(i, _):
            base = row0 - (i + 1) * 16
            rem = b_cnt - i * 16
            pltpu.sync_copy(
                h_hbm.at[pl.ds(pl.multiple_of(base * hd, 8), _LA * hd)], lahv)
            for j in range(16):
                @pl.when(16 - j <= rem)
                def _(j=j):
                    for kk in range(nk):
                        accv[pl.ds(kk * 16, 16)] = jnp.maximum(
                            accv[pl.ds(kk * 16, 16)],
                            lahv[pl.ds(j * hd + kk * 16, 16)])
            return 0

        lax.fori_loop(0, (b_cnt + 15) // 16, bfold_body, 0)


        def fill_rep(roff):
            regs = [accv[pl.ds(kk * 16, 16)] for kk in range(nk)]
            for t in range(16):
                for kk in range(nk):
                    repv[pl.ds(roff + t * hd + kk * 16, 16)] = regs[kk]

        def fire_rows(roff, dst_row, nrows):
            pltpu.async_copy(
                repv.at[pl.ds(roff, nrows * hd)],
                bc_hbm.at[pl.ds(pl.multiple_of(dst_row * hd, 8), nrows * hd)],
                semb)

        def drain_rows(k16, r1):
            def d16(i, _):
                pltpu.make_async_copy(
                    repv.at[pl.ds(0, 16 * hd)],
                    bc_hbm.at[pl.ds(0, 16 * hd)], semb).wait()
                return 0
            lax.fori_loop(0, k16, d16, 0)

            def d1(i, _):
                pltpu.make_async_copy(
                    repv.at[pl.ds(0, hd)],
                    bc_hbm.at[pl.ds(0, hd)], semb).wait()
                return 0
            lax.fori_loop(0, r1, d1, 0)

        def close_run(fin, rs, re, roff, fired, big):
            c = jnp.where(fin, re - rs, 0)

            @pl.when(fin)
            def _():
                fill_rep(roff)

            @pl.when(c >= 16)
            def _():
                fire_rows(roff, rs, 16)

            @pl.when(c >= 32)
            def _():
                fire_rows(roff, rs + 16, 16)

            @pl.when(c > 16)
            def _():
                fire_rows(roff, re - 16, 16)

            add = (jnp.where(c >= 16, 16, 0) + jnp.where(c >= 32, 16, 0)
                   + jnp.where(c > 16, 16, 0))
            pos = rs
            for szz in (8, 4, 2, 1):
                hit = jnp.logical_and(c < 16, (c & szz) > 0)

                @pl.when(hit)
                def _(pos=pos, szz=szz):
                    fire_rows(roff, pos, szz)

                pos = jnp.where(hit, pos + szz, pos)
                add = add + jnp.where(hit, szz, 0)

            isbig = c > 48
            big_rs, big_gap, big_roff = big
            big_rs = jnp.where(isbig, rs, big_rs)
            big_gap = jnp.where(isbig, c - 48, big_gap)
            big_roff = jnp.where(isbig, roff, big_roff)
            return fired + add, (big_rs, big_gap, big_roff)

        def make_group_body(p_off_id, p_off_h, gbase):
            def group_body(g, carry):
                (cur_id, slot, rs, fired, drained, prevf,
                 big_rs, big_gap, big_roff) = carry
                big = (big_rs, big_gap, big_roff)
                idvec = idv[pl.ds(p_off_id + g * 16, 16)]
                for j in range(16):
                    rid = idvec[j]
                    same = rid == cur_id
                    i = g * 16 + j
                    i_glob = gbase + g * 16 + j
                    fin = jnp.logical_not(same)
                    roff = pl.multiple_of(slot * rrow, 8)

                    @pl.when(same)
                    def _(i=i):
                        for kk in range(nk):
                            accv[pl.ds(kk * 16, 16)] = jnp.maximum(
                                accv[pl.ds(kk * 16, 16)],
                                hv[pl.ds(p_off_h + i * hd + kk * 16, 16)])

                    fired, big = close_run(fin, rs, i_glob, roff, fired, big)
                    slot = jnp.where(fin, (slot + 1) % _RB, slot)
                    rs = jnp.where(fin, i_glob, rs)

                    @pl.when(fin)
                    def _(i=i):
                        for kk in range(nk):
                            accv[pl.ds(kk * 16, 16)] = hv[
                                pl.ds(p_off_h + i * hd + kk * 16, 16)]

                    cur_id = jnp.where(same, cur_id, rid)

                big_rs, big_gap, big_roff = big
                broff = pl.multiple_of(big_roff, 8)

                def bigfix(t, _):
                    fire_rows(broff, big_rs + 32 + t * 16, 16)
                    return 0

                lax.fori_loop(0, big_gap // 16, bigfix, 0)

                @pl.when(jnp.logical_and(big_gap > 0, big_gap % 16 > 0))
                def _():
                    fire_rows(broff, big_rs + 48 + big_gap - 32, 16)

                fired = fired + (big_gap // 16) * 16 + jnp.where(
                    jnp.logical_and(big_gap > 0, big_gap % 16 > 0), 16, 0)
                big_rs = jnp.int32(0)
                big_gap = jnp.int32(0)
                big_roff = jnp.int32(0)

                need = prevf - drained
                drain_rows(need // 16, need % 16)
                drained = drained + need
                prevf = fired
                return (cur_id, slot, rs, fired, drained, prevf,
                        big_rs, big_gap, big_roff)
            return group_body

        def chunk_body(ch, carry):
            p = ch % 2
            p_off_id = pl.multiple_of(p * _CH, 16)
            p_off_h = pl.multiple_of(p * _CH * hd, 8)
            gbase = row0 + ch * _CH
            pltpu.make_async_copy(
                cl_hbm.at[pl.ds(0, _CH)], idv.at[0:_CH], semc).wait()
            pltpu.make_async_copy(
                h_hbm.at[pl.ds(0, _CH * hd)], hv.at[0:_CH * hd], semc).wait()

            @pl.when(ch + 1 < nch)
            def _():
                q = (ch + 1) % 2
                base2 = row0 + (ch + 1) * _CH
                pltpu.async_copy(
                    cl_hbm.at[pl.ds(pl.multiple_of(base2, 16), _CH)],
                    idv.at[pl.ds(pl.multiple_of(q * _CH, 16), _CH)], semc)
                pltpu.async_copy(
                    h_hbm.at[pl.ds(pl.multiple_of(base2 * hd, 8), _CH * hd)],
                    hv.at[pl.ds(pl.multiple_of(q * _CH * hd, 8), _CH * hd)],
                    semc)

            return lax.fori_loop(
                0, _CH // 16, make_group_body(p_off_id, p_off_h, gbase),
                carry)

        init_cur = jnp.where(continued, prev_id, jnp.int32(-1))
        (cur_id, slot, rs, fired, drained, prevf,
         _bg1, _bg2, _bg3) = lax.fori_loop(
            0, nch, chunk_body,
            (init_cur, jnp.int32(0), row0, jnp.int32(0), jnp.int32(0),
             jnp.int32(0), jnp.int32(0), jnp.int32(0), jnp.int32(0)))

        probe_base = jnp.minimum(rend, n - 64)
        pltpu.sync_copy(
            cl_hbm.at[pl.ds(pl.multiple_of(probe_base, 16), 64)], laidv)
        la_cnt = jnp.int32(0)
        still = rend < n
        for kg in range(4):
            valid = rend + kg * 16 + 16 <= n
            pc = _prefix_count(
                jnp.where(laidv[pl.ds(kg * 16, 16)] == cur_id, 1, 0))
            take = jnp.logical_and(still, valid)
            la_cnt = jnp.where(take, la_cnt + pc, la_cnt)
            still = jnp.logical_and(
                take, jnp.logical_and(pc == 16, rend + kg * 16 + 16 < n))

        trip_a = jnp.where(still, (n - rend - 64) // 16, 0)

        def ext_body(i, st):
            cnt, stl = st
            base = rend + 64 + i * 16
            pltpu.sync_copy(
                cl_hbm.at[pl.ds(pl.multiple_of(base, 16), 16)], bidv)
            pc = _prefix_count(jnp.where(bidv[...] == cur_id, 1, 0))
            cnt = jnp.where(stl, cnt + pc, cnt)
            stl = jnp.logical_and(stl, jnp.logical_and(pc == 16, base + 16 < n))
            return (cnt, stl)

        la_cnt, _ = lax.fori_loop(0, trip_a, ext_body, (la_cnt, still))

        def lb_body(i, _):
            base = rend + i * 16
            rem = la_cnt - i * 16
            pltpu.sync_copy(
                h_hbm.at[pl.ds(pl.multiple_of(base * hd, 8), _LA * hd)], lahv)
            for j in range(16):
                @pl.when(j < rem)
                def _(j=j):
                    for kk in range(nk):
                        accv[pl.ds(kk * 16, 16)] = jnp.maximum(
                            accv[pl.ds(kk * 16, 16)],
                            lahv[pl.ds(j * hd + kk * 16, 16)])
            return 0

        lax.fori_loop(0, (la_cnt + 15) // 16, lb_body, 0)

        roff = pl.multiple_of(slot * rrow, 8)
        fired, big = close_run(jnp.bool_(True), rs, rend, roff, fired,
                               (jnp.int32(0), jnp.int32(0), jnp.int32(0)))
        big_rs, big_gap, big_roff = big
        broff = pl.multiple_of(big_roff, 8)

        def bigfix(t, _):
            fire_rows(broff, big_rs + 32 + t * 16, 16)
            return 0

        lax.fori_loop(0, big_gap // 16, bigfix, 0)

        @pl.when(jnp.logical_and(big_gap > 0, big_gap % 16 > 0))
        def _():
            fire_rows(broff, big_rs + 48 + big_gap - 32, 16)

        fired = fired + (big_gap // 16) * 16 + jnp.where(
            jnp.logical_and(big_gap > 0, big_gap % 16 > 0), 16, 0)

        need = fired - drained
        drain_rows(need // 16, need % 16)

    return k(hmat.reshape(-1), cl).reshape(n, hd)


def _seg_max(hmat, cl):
    n, hd = hmat.shape
    r_per = n // _NW
    nk = hd // 16
    nch = r_per // _CH

    @functools.partial(
        pl.kernel,
        out_type=jax.ShapeDtypeStruct((_C * hd,), jnp.float32),
        mesh=_sc_mesh(),
        scratch_types=[
            pltpu.VMEM((2 * _CH,), jnp.int32),
            pltpu.VMEM((2 * _CH * hd,), jnp.float32),
            pltpu.VMEM((_RS * hd,), jnp.float32),
            pltpu.VMEM((_ZB * hd,), jnp.float32),
            pltpu.VMEM((16,), jnp.int32),
            pltpu.VMEM((16,), jnp.int32),
            pltpu.VMEM((64,), jnp.int32),
            pltpu.VMEM((_LA * hd,), jnp.float32),
            pltpu.SemaphoreType.DMA,
            pltpu.SemaphoreType.DMA,
        ],
    )
    def k(h_hbm, cl_hbm, agg_hbm, idv, hv, accr, zv, bidv, bidv2, laidv, lahv,
          semc, semf):
        wid = lax.axis_index("s") * _NC + lax.axis_index("c")
        row0 = wid * r_per

        pltpu.sync_copy(
            cl_hbm.at[pl.ds(pl.multiple_of(jnp.maximum(row0 - 16, 0), 16), 16)],
            bidv)
        prev_id = jnp.where(wid > 0, bidv[...][15], -1)
        nxt_base = jnp.minimum(row0 + r_per, n - 16)
        pltpu.sync_copy(
            cl_hbm.at[pl.ds(pl.multiple_of(nxt_base, 16), 16)], bidv2)
        a_hi = jnp.where(wid < _NW - 1, bidv2[...][0], _C)
        pltpu.sync_copy(cl_hbm.at[pl.ds(pl.multiple_of(row0, 16), 16)], bidv)
        first_id = bidv[...][0]

        pltpu.async_copy(
            cl_hbm.at[pl.ds(pl.multiple_of(row0, 16), _CH)],
            idv.at[0:_CH], semc)
        pltpu.async_copy(
            h_hbm.at[pl.ds(pl.multiple_of(row0 * hd, 8), _CH * hd)],
            hv.at[0:_CH * hd], semc)

        def zfill(j, _):
            zv[pl.ds(j * 16, 16)] = jnp.zeros((16,), jnp.float32)
            return 0
        lax.fori_loop(0, (_ZB * hd) // 16, zfill, 0)

        z0 = jnp.where(wid == 0, 0,
                       first_id + jnp.where(prev_id == first_id, 1, 0))
        zcnt = jnp.maximum(a_hi - z0, 0)
        pos = z0
        for sz in (_ZB, 16, 1):
            def zbody(i, p, sz=sz):
                pltpu.sync_copy(
                    zv.at[0:sz * hd],
                    agg_hbm.at[pl.ds(pl.multiple_of(p * hd, 8), sz * hd)])
                return p + sz
            cnt = zcnt // sz
            pos = lax.fori_loop(0, cnt, zbody, pos)
            zcnt = zcnt - cnt * sz

        def fire(cur_id, aoff):
            pltpu.async_copy(
                accr.at[pl.ds(aoff, hd)],
                agg_hbm.at[pl.ds(pl.multiple_of(cur_id * hd, 8), hd)],
                semf)

        def drain1():
            pltpu.make_async_copy(
                accr.at[pl.ds(0, hd)], agg_hbm.at[pl.ds(0, hd)], semf).wait()

        def make_group_body(p_off_id, p_off_h):
            def group_body(g, carry):
                cur_id, own, slot, pend = carry
                idvec = idv[pl.ds(p_off_id + g * 16, 16)]
                for j in range(16):
                    rid = idvec[j]
                    same = rid == cur_id
                    i = g * 16 + j
                    fin = jnp.logical_and(jnp.logical_not(same), own == 1)
                    aoff = pl.multiple_of(slot * hd, 8)

                    @pl.when(jnp.logical_and(same, own == 1))
                    def _(i=i, aoff=aoff):
                        for kk in range(nk):
                            accr[pl.ds(aoff + kk * 16, 16)] = jnp.maximum(
                                accr[pl.ds(aoff + kk * 16, 16)],
                                hv[pl.ds(p_off_h + i * hd + kk * 16, 16)])

                    @pl.when(fin)
                    def _(cur_id=cur_id, aoff=aoff):
                        fire(cur_id, aoff)

                    slot = jnp.where(fin, (slot + 1) % _RS, slot)
                    pend = jnp.where(fin, pend + 1, pend)
                    do_drain = pend > _RS - 1

                    @pl.when(do_drain)
                    def _():
                        drain1()

                    pend = jnp.where(do_drain, pend - 1, pend)
                    aoff2 = pl.multiple_of(slot * hd, 8)

                    @pl.when(jnp.logical_not(same))
                    def _(i=i, aoff2=aoff2):
                        for kk in range(nk):
                            accr[pl.ds(aoff2 + kk * 16, 16)] = hv[
                                pl.ds(p_off_h + i * hd + kk * 16, 16)]

                    cur_id = jnp.where(same, cur_id, rid)
                    own = jnp.where(same, own, 1)
                return (cur_id, own, slot, pend)
            return group_body

        def chunk_body(ch, carry):
            p = ch % 2
            p_off_id = pl.multiple_of(p * _CH, 16)
            p_off_h = pl.multiple_of(p * _CH * hd, 8)
            pltpu.make_async_copy(
                cl_hbm.at[pl.ds(0, _CH)], idv.at[0:_CH], semc).wait()
            pltpu.make_async_copy(
                h_hbm.at[pl.ds(0, _CH * hd)], hv.at[0:_CH * hd], semc).wait()

            @pl.when(ch + 1 < nch)
            def _():
                q = (ch + 1) % 2
                base2 = row0 + (ch + 1) * _CH
                pltpu.async_copy(
                    cl_hbm.at[pl.ds(pl.multiple_of(base2, 16), _CH)],
                    idv.at[pl.ds(pl.multiple_of(q * _CH, 16), _CH)], semc)
                pltpu.async_copy(
                    h_hbm.at[pl.ds(pl.multiple_of(base2 * hd, 8), _CH * hd)],
                    hv.at[pl.ds(pl.multiple_of(q * _CH * hd, 8), _CH * hd)],
                    semc)

            return lax.fori_loop(
                0, _CH // 16, make_group_body(p_off_id, p_off_h), carry)

        cur_id, own, slot, pend = lax.fori_loop(
            0, nch, chunk_body,
            (prev_id, jnp.int32(0), jnp.int32(0), jnp.int32(0)))

        rend = row0 + r_per
        probe_base = jnp.minimum(rend, n - 64)
        pltpu.sync_copy(
            cl_hbm.at[pl.ds(pl.multiple_of(probe_base, 16), 64)], laidv)
        la_cnt = jnp.int32(0)
        still = own == 1
        for kg in range(4):
            valid = rend + kg * 16 + 16 <= n
            pc = _prefix_count(
                jnp.where(laidv[pl.ds(kg * 16, 16)] == cur_id, 1, 0))
            take = jnp.logical_and(still, valid)
            la_cnt = jnp.where(take, la_cnt + pc, la_cnt)
            still = jnp.logical_and(
                take, jnp.logical_and(pc == 16, rend + kg * 16 + 16 < n))

        trip_a = jnp.where(still, (n - rend - 64) // 16, 0)

        def ext_body(i, st):
            cnt, stl = st
            base = rend + 64 + i * 16
            pltpu.sync_copy(
                cl_hbm.at[pl.ds(pl.multiple_of(base, 16), 16)], bidv)
            pc = _prefix_count(jnp.where(bidv[...] == cur_id, 1, 0))
            cnt = jnp.where(stl, cnt + pc, cnt)
            stl = jnp.logical_and(stl, jnp.logical_and(pc == 16, base + 16 < n))
            return (cnt, stl)

        la_cnt, _ = lax.fori_loop(0, trip_a, ext_body, (la_cnt, still))

        aofff = pl.multiple_of(slot * hd, 8)

        def lb_body(i, _):
            base = rend + i * 16
            rem = la_cnt - i * 16
            pltpu.sync_copy(
                h_hbm.at[pl.ds(pl.multiple_of(base * hd, 8), _LA * hd)], lahv)
            for j in range(16):
                @pl.when(j < rem)
                def _(j=j):
                    for kk in range(nk):
                        accr[pl.ds(aofff + kk * 16, 16)] = jnp.maximum(
                            accr[pl.ds(aofff + kk * 16, 16)],
                            lahv[pl.ds(j * hd + kk * 16, 16)])
            return 0

        lax.fori_loop(0, (la_cnt + 15) // 16, lb_body, 0)

        @pl.when(own == 1)
        def _():
            fire(cur_id, aofff)

        pend = jnp.where(own == 1, pend + 1, pend)

        def drain_body(i, _):
            drain1()
            return 0

        lax.fori_loop(0, pend, drain_body, 0)

    return k(hmat.reshape(-1), cl).reshape(_C, hd)


def kernel(x, cluster, edge_index, time_step_len,
           W0, b0, g0, be0, W1, b1, g1, be1, W2, b2, g2, be2, Wf, bf):
    hd = W0.shape[1]

    h = _mlp_first(x, W0, b0, g0, be0)
    for (W, b, g, be) in ((W1, b1, g1, be1), (W2, b2, g2, be2)):
        bc = _seg_bcast(h, cluster)
        h = _mlp_pair(h, bc, W[:hd], W[hd:], b, g, be, ln_relu=True)
    bc = _seg_bcast(h, cluster)
    hf = _mlp_pair(h, bc, Wf[:hd], Wf[hd:], bf, bf, bf, ln_relu=False)
    aggf = _seg_max(hf, cluster)
    return _normalize(aggf)
